# Initial kernel scaffold; baseline (speedup 1.0000x reference)
import jax
import jax.numpy as jnp
from jax.experimental import pallas as pl


def kernel(x, edge_index, W, att_src, att_dst, bias):
    raise NotImplementedError("write your pallas kernel here")


# speedup vs baseline: 28.1754x; 28.1754x over previous
import functools

import jax
import jax.numpy as jnp
from jax import lax
from jax.experimental import pallas as pl
from jax.experimental.pallas import tpu as pltpu
from jax.experimental.pallas import tpu_sc as plsc

N = 10000
E = 320000
D = 128
NC = 2
NS = 16
NW = NC * NS
EC = 128
NCHUNK = E // EC
KMAX = -(-NCHUNK // NW)
WCH = 128
NWC = N // WCH
NTAIL = N - NWC * WCH


def _tc_pre_body(x_ref, w_ref, as_ref, ad_ref, xl_ref, sv_ref, dv_ref, es_ref):
    xl = jnp.dot(x_ref[...], w_ref[...], preferred_element_type=jnp.float32)
    xl_ref[...] = xl
    sv = jnp.sum(xl * as_ref[...], axis=1, keepdims=True)
    dv = jnp.sum(xl * ad_ref[...], axis=1, keepdims=True)
    sv_ref[...] = sv
    dv_ref[...] = dv
    z = sv + dv
    es_ref[...] = jnp.exp(jnp.maximum(z, 0.2 * z))


def _sc_body(xl, a_s, a_d, src, dst, pout, pden,
             as_v, ad_v, srcv, dstv, eav, rows_v, dstg_v,
             acc_sh, den_sh, sem):
    c = lax.axis_index("c")
    s = lax.axis_index("s")
    w = s * NC + c
    zero16 = jnp.zeros((16,), jnp.float32)

    pltpu.sync_copy(a_s, as_v)
    pltpu.sync_copy(a_d, ad_v)

    def zs_body(r, _):
        for j in range(8):
            rows_v[r, pl.ds(16 * j, 16)] = zero16
        return 0
    lax.fori_loop(0, WCH, zs_body, 0)

    def zd_body(i, _):
        dstg_v[pl.ds(i * 16, 16)] = zero16
        return 0
    lax.fori_loop(0, 1000 // 16, zd_body, 0)

    for q in range(-(-NWC // NS)):
        ch = s + q * NS

        @pl.when(ch < NWC)
        def _():
            pltpu.sync_copy(rows_v, acc_sh.at[pl.ds(ch * WCH, WCH)])

    @pl.when(s == 0)
    def _():
        pltpu.sync_copy(rows_v.at[pl.ds(0, NTAIL)],
                        acc_sh.at[pl.ds(NWC * WCH, NTAIL)])

    @pl.when(s < 10)
    def _():
        pltpu.sync_copy(dstg_v, den_sh.at[pl.ds(s * 1000, 1000)])

    plsc.subcore_barrier()

    def chunk_body(k, _):
        cidx = w + k * NW

        @pl.when(cidx < NCHUNK)
        def _():
            base = cidx * EC
            pltpu.sync_copy(src.at[pl.ds(base, EC)], srcv)
            pltpu.sync_copy(dst.at[pl.ds(base, EC)], dstv)
            cp = pltpu.async_copy(xl.at[srcv], rows_v, sem)
            for j in range(8):
                si = srcv[pl.ds(16 * j, 16)]
                di = dstv[pl.ds(16 * j, 16)]
                z = plsc.load_gather(as_v, [si]) + plsc.load_gather(ad_v, [di])
                eav[pl.ds(16 * j, 16)] = jnp.exp(jnp.maximum(z, 0.2 * z))
            cp.wait()

            def scale_body(g, _):
                ea16 = eav[pl.ds(16 * g, 16)]
                for i in range(16):
                    e = ea16[i]
                    r = 16 * g + i
                    for j in range(8):
                        rows_v[r, pl.ds(16 * j, 16)] = (
                            rows_v[r, pl.ds(16 * j, 16)] * e)
                return 0
            lax.fori_loop(0, EC // 16, scale_body, 0)

            pltpu.sync_copy(rows_v, acc_sh.at[dstv], add=True)
            pltpu.sync_copy(eav, den_sh.at[dstv], add=True)
        return 0
    lax.fori_loop(0, KMAX, chunk_body, 0)

    plsc.subcore_barrier()

    for q in range(-(-NWC // NS)):
        ch = s + q * NS

        @pl.when(ch < NWC)
        def _():
            r0 = ch * WCH
            pltpu.sync_copy(acc_sh.at[pl.ds(r0, WCH)], rows_v)
            pltpu.sync_copy(rows_v, pout.at[pl.ds(c * N + r0, WCH)])

    @pl.when(s == 0)
    def _():
        r0 = NWC * WCH
        pltpu.sync_copy(acc_sh.at[pl.ds(r0, NTAIL)], rows_v.at[pl.ds(0, NTAIL)])
        pltpu.sync_copy(rows_v.at[pl.ds(0, NTAIL)], pout.at[pl.ds(c * N + r0, NTAIL)])

    @pl.when(s < 10)
    def _():
        pltpu.sync_copy(den_sh.at[pl.ds(s * 1000, 1000)], dstg_v)
        pltpu.sync_copy(dstg_v, pden.at[pl.ds(c * N + s * 1000, 1000)])


def _tc_post_body(p_ref, d_ref, xl_ref, x_ref, es_ref, b_ref, o_ref):
    es = es_ref[...]
    num = p_ref[0] + p_ref[1] + es * xl_ref[...]
    den = d_ref[0] + d_ref[1] + es + 1e-16
    o_ref[...] = num / den + b_ref[...] + x_ref[...]


_BR = 1000


@jax.jit
def kernel(x, edge_index, W, att_src, att_dst, bias):
    grid = (N // _BR,)
    xl, a_s, a_d, es = pl.pallas_call(
        _tc_pre_body,
        grid=grid,
        in_specs=[
            pl.BlockSpec((_BR, D), lambda i: (i, 0)),
            pl.BlockSpec((D, D), lambda i: (0, 0)),
            pl.BlockSpec((1, D), lambda i: (0, 0)),
            pl.BlockSpec((1, D), lambda i: (0, 0)),
        ],
        out_specs=[
            pl.BlockSpec((_BR, D), lambda i: (i, 0)),
            pl.BlockSpec((_BR, 1), lambda i: (i, 0)),
            pl.BlockSpec((_BR, 1), lambda i: (i, 0)),
            pl.BlockSpec((_BR, 1), lambda i: (i, 0)),
        ],
        out_shape=[
            jax.ShapeDtypeStruct((N, D), jnp.float32),
            jax.ShapeDtypeStruct((N, 1), jnp.float32),
            jax.ShapeDtypeStruct((N, 1), jnp.float32),
            jax.ShapeDtypeStruct((N, 1), jnp.float32),
        ],
    )(x, W, att_src, att_dst)

    sc = pl.kernel(
        _sc_body,
        out_type=[
            jax.ShapeDtypeStruct((NC * N, D), jnp.float32),
            jax.ShapeDtypeStruct((NC * N,), jnp.float32),
        ],
        mesh=plsc.VectorSubcoreMesh(core_axis_name="c", subcore_axis_name="s"),
        compiler_params=pltpu.CompilerParams(needs_layout_passes=False),
        scratch_types=[
            pltpu.VMEM((N,), jnp.float32),
            pltpu.VMEM((N,), jnp.float32),
            pltpu.VMEM((EC,), jnp.int32),
            pltpu.VMEM((EC,), jnp.int32),
            pltpu.VMEM((EC,), jnp.float32),
            pltpu.VMEM((EC, D), jnp.float32),
            pltpu.VMEM((1000,), jnp.float32),
            pltpu.VMEM_SHARED((N, D), jnp.float32),
            pltpu.VMEM_SHARED((N,), jnp.float32),
            pltpu.SemaphoreType.DMA,
        ],
    )
    pout, pden = sc(xl, a_s.reshape(N), a_d.reshape(N),
                    edge_index[0], edge_index[1])

    out = pl.pallas_call(
        _tc_post_body,
        grid=grid,
        in_specs=[
            pl.BlockSpec((NC, _BR, D), lambda i: (0, i, 0)),
            pl.BlockSpec((NC, _BR, 1), lambda i: (0, i, 0)),
            pl.BlockSpec((_BR, D), lambda i: (i, 0)),
            pl.BlockSpec((_BR, D), lambda i: (i, 0)),
            pl.BlockSpec((_BR, 1), lambda i: (i, 0)),
            pl.BlockSpec((1, D), lambda i: (0, 0)),
        ],
        out_specs=pl.BlockSpec((_BR, D), lambda i: (i, 0)),
        out_shape=jax.ShapeDtypeStruct((N, D), jnp.float32),
    )(pout.reshape(NC, N, D), pden.reshape(NC, N, 1), xl, x, es, bias.reshape(1, D))
    return out


# speedup vs baseline: 35.9272x; 1.2751x over previous
import functools

import jax
import jax.numpy as jnp
from jax import lax
from jax.experimental import pallas as pl
from jax.experimental.pallas import tpu as pltpu
from jax.experimental.pallas import tpu_sc as plsc

N = 10000
E = 320000
D = 128
NC = 2
NS = 16
NW = NC * NS
EC = 80
EPT = E // NW
KCH = EPT // EC
WCH = EC
NWC = N // WCH


def _tc_pre_body(x_ref, w_ref, as_ref, ad_ref, xl_ref, sv_ref, dv_ref, es_ref):
    xl = jnp.dot(x_ref[...], w_ref[...], preferred_element_type=jnp.float32)
    xl_ref[...] = xl
    sv = jnp.sum(xl * as_ref[...], axis=1, keepdims=True)
    dv = jnp.sum(xl * ad_ref[...], axis=1, keepdims=True)
    sv_ref[...] = sv
    dv_ref[...] = dv
    z = sv + dv
    es_ref[...] = jnp.exp(jnp.maximum(z, 0.2 * z))


def _sc_body(xl, a_s, a_d, src, dst, pout, pden,
             as_v, ad_v, src0, src1, dst0, dst1, dsc0, dsc1, eav,
             rows0, rows1, dstg_v, acc_sh, den_sh, sem_g, sem_i, sem_s):
    c = lax.axis_index("c")
    s = lax.axis_index("s")
    w = s * NC + c
    zero16 = jnp.zeros((16,), jnp.float32)

    pltpu.sync_copy(a_s, as_v)
    pltpu.sync_copy(a_d, ad_v)

    def zs_body(r, _):
        for j in range(8):
            rows0[r, pl.ds(16 * j, 16)] = zero16
        return 0
    lax.fori_loop(0, EC, zs_body, 0)

    def zd_body(i, _):
        dstg_v[pl.ds(i * 16, 16)] = zero16
        return 0
    lax.fori_loop(0, 1000 // 16, zd_body, 0)

    for q in range(-(-NWC // NS)):
        ch = s + q * NS

        @pl.when(ch < NWC)
        def _():
            pltpu.sync_copy(rows0, acc_sh.at[pl.ds(ch * WCH, WCH)])

    @pl.when(s < 10)
    def _():
        pltpu.sync_copy(dstg_v, den_sh.at[pl.ds(s * 1000, 1000)])

    plsc.subcore_barrier()

    eb = w * EPT
    srcs = (src0, src1)
    dsts = (dst0, dst1)
    dscs = (dsc0, dsc1)
    rows = (rows0, rows1)

    def fire_idx(k, o):
        base = eb + k * EC
        pltpu.async_copy(src.at[pl.ds(base, EC)], srcs[o], sem_i)
        pltpu.async_copy(dst.at[pl.ds(base, EC)], dsts[o], sem_i)

    def drain_idx(o):
        pltpu.make_async_copy(src.at[pl.ds(0, EC)], srcs[o], sem_i).wait()
        pltpu.make_async_copy(dst.at[pl.ds(0, EC)], dsts[o], sem_i).wait()

    def step(k, b, first=False, last=False):
        o = 1 - b
        pltpu.make_async_copy(xl.at[pl.ds(0, EC)], rows[b], sem_g).wait()
        for g in range(EC // 16):
            si = srcs[b][pl.ds(16 * g, 16)]
            di = dsts[b][pl.ds(16 * g, 16)]
            z = plsc.load_gather(as_v, [si]) + plsc.load_gather(ad_v, [di])
            eav[pl.ds(16 * g, 16)] = jnp.exp(jnp.maximum(z, 0.2 * z))
            dscs[b][pl.ds(16 * g, 16)] = di
        if not last:
            fire_idx(k + 1, o)

        def scale_body(g, _):
            ea16 = eav[pl.ds(16 * g, 16)]
            for i in range(16):
                e = ea16[i]
                r = 16 * g + i
                for jj in range(8):
                    rows[b][r, pl.ds(16 * jj, 16)] = (
                        rows[b][r, pl.ds(16 * jj, 16)] * e)
            return 0
        lax.fori_loop(0, EC // 16, scale_body, 0)

        if not first:
            pltpu.make_async_copy(xl.at[pl.ds(0, EC)], rows[o], sem_s).wait()
        if not last:
            drain_idx(o)
            pltpu.async_copy(xl.at[srcs[o]], rows[o], sem_g)
        pltpu.async_copy(rows[b], acc_sh.at[dscs[b]], sem_s, add=True)
        pltpu.sync_copy(eav, den_sh.at[dscs[b]], add=True)

    fire_idx(0, 0)
    drain_idx(0)
    pltpu.async_copy(xl.at[srcs[0]], rows[0], sem_g)
    step(0, 0, first=True)

    def body2(j, _):
        step(2 * j + 1, 1)
        step(2 * j + 2, 0)
        return 0
    lax.fori_loop(0, (KCH - 3) // 2, body2, 0)

    step(KCH - 2, 1)
    step(KCH - 1, 0, last=True)
    pltpu.make_async_copy(xl.at[pl.ds(0, EC)], rows[0], sem_s).wait()

    plsc.subcore_barrier()

    for q in range(-(-NWC // NS)):
        ch = s + q * NS

        @pl.when(ch < NWC)
        def _():
            r0 = ch * WCH
            pltpu.sync_copy(acc_sh.at[pl.ds(r0, WCH)], rows0)
            pltpu.sync_copy(rows0, pout.at[pl.ds(c * N + r0, WCH)])

    @pl.when(s < 10)
    def _():
        pltpu.sync_copy(den_sh.at[pl.ds(s * 1000, 1000)], dstg_v)
        pltpu.sync_copy(dstg_v, pden.at[pl.ds(c * N + s * 1000, 1000)])


def _tc_post_body(p_ref, d_ref, xl_ref, x_ref, es_ref, b_ref, o_ref):
    es = es_ref[...]
    num = p_ref[0] + p_ref[1] + es * xl_ref[...]
    den = d_ref[0] + d_ref[1] + es + 1e-16
    o_ref[...] = num / den + b_ref[...] + x_ref[...]


_BR = 1000


@jax.jit
def kernel(x, edge_index, W, att_src, att_dst, bias):
    grid = (N // _BR,)
    xl, a_s, a_d, es = pl.pallas_call(
        _tc_pre_body,
        grid=grid,
        in_specs=[
            pl.BlockSpec((_BR, D), lambda i: (i, 0)),
            pl.BlockSpec((D, D), lambda i: (0, 0)),
            pl.BlockSpec((1, D), lambda i: (0, 0)),
            pl.BlockSpec((1, D), lambda i: (0, 0)),
        ],
        out_specs=[
            pl.BlockSpec((_BR, D), lambda i: (i, 0)),
            pl.BlockSpec((_BR, 1), lambda i: (i, 0)),
            pl.BlockSpec((_BR, 1), lambda i: (i, 0)),
            pl.BlockSpec((_BR, 1), lambda i: (i, 0)),
        ],
        out_shape=[
            jax.ShapeDtypeStruct((N, D), jnp.float32),
            jax.ShapeDtypeStruct((N, 1), jnp.float32),
            jax.ShapeDtypeStruct((N, 1), jnp.float32),
            jax.ShapeDtypeStruct((N, 1), jnp.float32),
        ],
    )(x, W, att_src, att_dst)

    sc = pl.kernel(
        _sc_body,
        out_type=[
            jax.ShapeDtypeStruct((NC * N, D), jnp.float32),
            jax.ShapeDtypeStruct((NC * N,), jnp.float32),
        ],
        mesh=plsc.VectorSubcoreMesh(core_axis_name="c", subcore_axis_name="s"),
        compiler_params=pltpu.CompilerParams(needs_layout_passes=False),
        scratch_types=[
            pltpu.VMEM((N,), jnp.float32),
            pltpu.VMEM((N,), jnp.float32),
            pltpu.VMEM((EC,), jnp.int32),
            pltpu.VMEM((EC,), jnp.int32),
            pltpu.VMEM((EC,), jnp.int32),
            pltpu.VMEM((EC,), jnp.int32),
            pltpu.VMEM((EC,), jnp.int32),
            pltpu.VMEM((EC,), jnp.int32),
            pltpu.VMEM((EC,), jnp.float32),
            pltpu.VMEM((EC, D), jnp.float32),
            pltpu.VMEM((EC, D), jnp.float32),
            pltpu.VMEM((1000,), jnp.float32),
            pltpu.VMEM_SHARED((N, D), jnp.float32),
            pltpu.VMEM_SHARED((N,), jnp.float32),
            pltpu.SemaphoreType.DMA,
            pltpu.SemaphoreType.DMA,
            pltpu.SemaphoreType.DMA,
        ],
    )
    pout, pden = sc(xl, a_s.reshape(N), a_d.reshape(N),
                    edge_index[0], edge_index[1])

    out = pl.pallas_call(
        _tc_post_body,
        grid=grid,
        in_specs=[
            pl.BlockSpec((NC, _BR, D), lambda i: (0, i, 0)),
            pl.BlockSpec((NC, _BR, 1), lambda i: (0, i, 0)),
            pl.BlockSpec((_BR, D), lambda i: (i, 0)),
            pl.BlockSpec((_BR, D), lambda i: (i, 0)),
            pl.BlockSpec((_BR, 1), lambda i: (i, 0)),
            pl.BlockSpec((1, D), lambda i: (0, 0)),
        ],
        out_specs=pl.BlockSpec((_BR, D), lambda i: (i, 0)),
        out_shape=jax.ShapeDtypeStruct((N, D), jnp.float32),
    )(pout.reshape(NC, N, D), pden.reshape(NC, N, 1), xl, x, es, bias.reshape(1, D))
    return out
